# baseline (speedup 1.0000x reference)
import functools

import jax
import jax.numpy as jnp
import numpy as np
from jax import lax
from jax.experimental import pallas as pl
from jax.experimental.pallas import tpu as pltpu
from jax.experimental.pallas import tpu_sc as plsc

N = 10000
E = 320000
NC = 2
NS = 16
L = 16
NW = NC * NS
EPW = E // NW
UNROLL = 5


def _mesh():
    return plsc.VectorSubcoreMesh(
        core_axis_name="c", subcore_axis_name="s", num_cores=NC, num_subcores=NS
    )


def _zero_fill(ref, nwords):
    zeros = jnp.zeros((L,), jnp.float32)

    def zbody(i):
        ref[pl.ds(i * L, L)] = zeros

    plsc.parallel_loop(0, nwords // L, 1, unroll=5)(zbody)


@functools.partial(
    pl.kernel,
    out_type=jax.ShapeDtypeStruct((NW, N), jnp.float32),
    mesh=_mesh(),
    scratch_types=[
        pltpu.VMEM((EPW,), jnp.int32),
        pltpu.VMEM((N,), jnp.float32),
        pltpu.SemaphoreType.DMA,
    ],
    compiler_params=pltpu.CompilerParams(needs_layout_passes=False),
)
def _deg_kernel(ei_hbm, out_hbm, dst_v, acc_v, sem1):
    c = lax.axis_index("c")
    s = lax.axis_index("s")
    wid = s * NC + c
    cp1 = pltpu.async_copy(ei_hbm.at[pl.ds(E + wid * EPW, EPW)], dst_v, sem1)
    _zero_fill(acc_v, N)
    cp1.wait()
    ones = jnp.ones((L,), jnp.float32)

    def ebody(i):
        d = dst_v[pl.ds(i * L, L)]
        plsc.addupdate_scatter(acc_v, [d], ones)

    plsc.parallel_loop(0, EPW // L, 1, unroll=UNROLL)(ebody)
    pltpu.sync_copy(acc_v, out_hbm.at[wid])


def _make_edge_pass(F):
    @functools.partial(
        pl.kernel,
        out_type=jax.ShapeDtypeStruct((NW, F * N), jnp.float32),
        mesh=_mesh(),
        scratch_types=[
            pltpu.VMEM((EPW,), jnp.int32),
            pltpu.VMEM((EPW,), jnp.int32),
            pltpu.VMEM((F * N,), jnp.float32),
            pltpu.VMEM((F * N,), jnp.float32),
            pltpu.VMEM_SHARED((F * N,), jnp.float32),
            pltpu.SemaphoreType.DMA,
            pltpu.SemaphoreType.DMA,
        ],
        compiler_params=pltpu.CompilerParams(needs_layout_passes=False),
    )
    def edge_kernel(ei_hbm, g_hbm, out_hbm,
                    src_v, dst_v, g_v, acc_v, g_sh, sem1, sem2):
        c = lax.axis_index("c")
        s = lax.axis_index("s")
        wid = s * NC + c
        cp1 = pltpu.async_copy(ei_hbm.at[pl.ds(wid * EPW, EPW)], src_v, sem1)
        cp2 = pltpu.async_copy(ei_hbm.at[pl.ds(E + wid * EPW, EPW)], dst_v, sem2)

        @pl.when(s == 0)
        def _():
            pltpu.sync_copy(g_hbm, g_sh)

        _zero_fill(acc_v, F * N)
        plsc.subcore_barrier()
        pltpu.sync_copy(g_sh, g_v)
        cp1.wait()
        cp2.wait()

        def ebody(i):
            off = i * L
            sv = src_v[pl.ds(off, L)]
            dv = dst_v[pl.ds(off, L)]
            for f in range(F):
                vals = plsc.load_gather(g_v, [sv + f * N])
                plsc.addupdate_scatter(acc_v, [dv + f * N], vals)

        plsc.parallel_loop(0, EPW // L, 1, unroll=UNROLL)(ebody)
        pltpu.sync_copy(acc_v, out_hbm.at[wid])

    return edge_kernel


_edge4 = _make_edge_pass(4)
_edge2 = _make_edge_pass(2)


def _tc1_body(degp_ref, x_ref, w1_ref, b1_ref, g1_ref, p_ref, pm_ref):
    deg = jnp.sum(degp_ref[...], axis=0) + 1.0
    p = lax.rsqrt(deg)
    pm = p / deg
    h = lax.dot_general(
        w1_ref[...], x_ref[...], (((0,), (1,)), ((), ())),
        preferred_element_type=jnp.float32,
    )
    h = h + b1_ref[...][:, None]
    g1_ref[...] = h * p[None, :]
    p_ref[...] = p
    pm_ref[...] = pm


_tc1 = pl.pallas_call(
    _tc1_body,
    out_shape=[
        jax.ShapeDtypeStruct((4, N), jnp.float32),
        jax.ShapeDtypeStruct((N,), jnp.float32),
        jax.ShapeDtypeStruct((N,), jnp.float32),
    ],
)


def _parts_acc(parts_ref, fin):
    acc1d = jnp.sum(parts_ref[...], axis=0)
    return jnp.stack(
        [acc1d[f * N:(f + 1) * N] for f in range(fin)], axis=0
    )


def _tcmid_body(fin, parts_ref, g_ref, pm_ref, p_ref, w_ref, b_ref, gnext_ref):
    acc = _parts_acc(parts_ref, fin) + g_ref[...]
    t = jnp.tanh(acc * pm_ref[...][None, :])
    h = lax.dot_general(
        w_ref[...], t, (((0,), (0,)), ((), ())),
        preferred_element_type=jnp.float32,
    ) + b_ref[...][:, None]
    gnext_ref[...] = h * p_ref[...][None, :]


def _make_tcmid(fin, fout):
    return pl.pallas_call(
        functools.partial(_tcmid_body, fin),
        out_shape=jax.ShapeDtypeStruct((fout, N), jnp.float32),
    )


_tcmid4 = _make_tcmid(4, 4)
_tcmid2 = _make_tcmid(4, 2)


def _tc3_body(parts_ref, g_ref, p_ref, wc_ref, bc_ref, out_ref):
    acc = _parts_acc(parts_ref, 2) + g_ref[...]
    t = jnp.tanh(acc * p_ref[...][None, :])
    out_ref[...] = lax.dot_general(
        t, wc_ref[...], (((0,), (0,)), ((), ())),
        preferred_element_type=jnp.float32,
    ) + bc_ref[...][None, :]


_tc3 = pl.pallas_call(
    _tc3_body,
    out_shape=jax.ShapeDtypeStruct((N, 10), jnp.float32),
)


def kernel(x, edge_index, W1, b1, W2, b2, W3, b3, Wc, bc):
    ei = edge_index.astype(jnp.int32).reshape(2 * E)

    degp = _deg_kernel(ei)
    g1, p, pm = _tc1(degp, x, W1, b1)

    parts1 = _edge4(ei, g1.reshape(4 * N))
    g2 = _tcmid4(parts1, g1, pm, p, W2, b2)

    parts2 = _edge4(ei, g2.reshape(4 * N))
    g3 = _tcmid2(parts2, g2, pm, p, W3, b3)

    parts3 = _edge2(ei, g3.reshape(2 * N))
    out = _tc3(parts3, g3, p, Wc, bc)
    return out


# speedup vs baseline: 1.0150x; 1.0150x over previous
import functools

import jax
import jax.numpy as jnp
import numpy as np
from jax import lax
from jax.experimental import pallas as pl
from jax.experimental.pallas import tpu as pltpu
from jax.experimental.pallas import tpu_sc as plsc

N = 10000
E = 320000
NC = 2
NS = 16
L = 16
NW = NC * NS
EPW = E // NW
UNROLL = 5


def _mesh():
    return plsc.VectorSubcoreMesh(
        core_axis_name="c", subcore_axis_name="s", num_cores=NC, num_subcores=NS
    )


def _zero_fill(ref, nwords):
    zeros = jnp.zeros((L,), jnp.float32)

    def zbody(i):
        ref[pl.ds(i * L, L)] = zeros

    plsc.parallel_loop(0, nwords // L, 1, unroll=5)(zbody)


@functools.partial(
    pl.kernel,
    out_type=jax.ShapeDtypeStruct((NW, N), jnp.float32),
    mesh=_mesh(),
    scratch_types=[
        pltpu.VMEM((EPW,), jnp.int32),
        pltpu.VMEM((N,), jnp.float32),
        pltpu.SemaphoreType.DMA,
    ],
    compiler_params=pltpu.CompilerParams(needs_layout_passes=False),
)
def _deg_kernel(ei_hbm, out_hbm, dst_v, acc_v, sem1):
    c = lax.axis_index("c")
    s = lax.axis_index("s")
    wid = s * NC + c
    cp1 = pltpu.async_copy(ei_hbm.at[pl.ds(E + wid * EPW, EPW)], dst_v, sem1)
    _zero_fill(acc_v, N)
    cp1.wait()
    ones = jnp.ones((L,), jnp.float32)

    def ebody(i):
        d = dst_v[pl.ds(i * L, L)]
        plsc.addupdate_scatter(acc_v, [d], ones)

    plsc.parallel_loop(0, EPW // L, 1, unroll=UNROLL)(ebody)
    pltpu.sync_copy(acc_v, out_hbm.at[wid])


def _make_edge_pass(F):
    @functools.partial(
        pl.kernel,
        out_type=jax.ShapeDtypeStruct((NW, F * N), jnp.float32),
        mesh=_mesh(),
        scratch_types=[
            pltpu.VMEM((EPW,), jnp.int32),
            pltpu.VMEM((EPW,), jnp.int32),
            pltpu.VMEM((F * N,), jnp.float32),
            pltpu.VMEM((F * N,), jnp.float32),
            pltpu.VMEM_SHARED((F * N,), jnp.float32),
            pltpu.SemaphoreType.DMA,
            pltpu.SemaphoreType.DMA,
        ],
        compiler_params=pltpu.CompilerParams(needs_layout_passes=False),
    )
    def edge_kernel(ei_hbm, g_hbm, out_hbm,
                    src_v, dst_v, g_v, acc_v, g_sh, sem1, sem2):
        c = lax.axis_index("c")
        s = lax.axis_index("s")
        wid = s * NC + c
        cp1 = pltpu.async_copy(ei_hbm.at[pl.ds(wid * EPW, EPW)], src_v, sem1)
        cp2 = pltpu.async_copy(ei_hbm.at[pl.ds(E + wid * EPW, EPW)], dst_v, sem2)

        @pl.when(s == 0)
        def _():
            pltpu.sync_copy(g_hbm, g_sh)

        _zero_fill(acc_v, F * N)
        plsc.subcore_barrier()
        pltpu.sync_copy(g_sh, g_v)
        cp1.wait()
        cp2.wait()

        def ebody(i):
            off = i * L
            sv = src_v[pl.ds(off, L)]
            dv = dst_v[pl.ds(off, L)]
            for f in range(F):
                vals = plsc.load_gather(g_v, [sv + f * N])
                plsc.addupdate_scatter(acc_v, [dv + f * N], vals)

        plsc.parallel_loop(0, EPW // L, 1, unroll=UNROLL)(ebody)
        pltpu.sync_copy(acc_v, out_hbm.at[wid])

    return edge_kernel


_edge4 = _make_edge_pass(4)
_edge2 = _make_edge_pass(2)


def _tch_body(x_ref, w1_ref, b1_ref, h_ref):
    h_ref[...] = lax.dot_general(
        w1_ref[...], x_ref[...], (((0,), (1,)), ((), ())),
        preferred_element_type=jnp.float32,
    ) + b1_ref[...][:, None]


_tch = pl.pallas_call(
    _tch_body,
    out_shape=jax.ShapeDtypeStruct((4, N), jnp.float32),
)


def _tcs_body(degp_ref, h_ref, g1_ref, p_ref, pm_ref):
    deg = jnp.sum(degp_ref[...], axis=0) + 1.0
    p = lax.rsqrt(deg)
    g1_ref[...] = h_ref[...] * p[None, :]
    p_ref[...] = p
    pm_ref[...] = p / deg


_tcs = pl.pallas_call(
    _tcs_body,
    out_shape=[
        jax.ShapeDtypeStruct((4, N), jnp.float32),
        jax.ShapeDtypeStruct((N,), jnp.float32),
        jax.ShapeDtypeStruct((N,), jnp.float32),
    ],
)


def _parts_acc(parts_ref, fin):
    acc1d = jnp.sum(parts_ref[...], axis=0)
    return jnp.stack(
        [acc1d[f * N:(f + 1) * N] for f in range(fin)], axis=0
    )


def _tcmid_body(fin, parts_ref, g_ref, pm_ref, p_ref, w_ref, b_ref, gnext_ref):
    acc = _parts_acc(parts_ref, fin) + g_ref[...]
    t = jnp.tanh(acc * pm_ref[...][None, :])
    h = lax.dot_general(
        w_ref[...], t, (((0,), (0,)), ((), ())),
        preferred_element_type=jnp.float32,
    ) + b_ref[...][:, None]
    gnext_ref[...] = h * p_ref[...][None, :]


def _make_tcmid(fin, fout):
    return pl.pallas_call(
        functools.partial(_tcmid_body, fin),
        out_shape=jax.ShapeDtypeStruct((fout, N), jnp.float32),
    )


_tcmid4 = _make_tcmid(4, 4)
_tcmid2 = _make_tcmid(4, 2)


def _tc3_body(parts_ref, g_ref, p_ref, wc_ref, bc_ref, out_ref):
    acc = _parts_acc(parts_ref, 2) + g_ref[...]
    t = jnp.tanh(acc * p_ref[...][None, :])
    out_ref[...] = lax.dot_general(
        t, wc_ref[...], (((0,), (0,)), ((), ())),
        preferred_element_type=jnp.float32,
    ) + bc_ref[...][None, :]


_tc3 = pl.pallas_call(
    _tc3_body,
    out_shape=jax.ShapeDtypeStruct((N, 10), jnp.float32),
)


def kernel(x, edge_index, W1, b1, W2, b2, W3, b3, Wc, bc):
    ei = edge_index.astype(jnp.int32).reshape(2 * E)

    h1 = _tch(x, W1, b1)
    degp = _deg_kernel(ei)
    g1, p, pm = _tcs(degp, h1)

    parts1 = _edge4(ei, g1.reshape(4 * N))
    g2 = _tcmid4(parts1, g1, pm, p, W2, b2)

    parts2 = _edge4(ei, g2.reshape(4 * N))
    g3 = _tcmid2(parts2, g2, pm, p, W3, b3)

    parts3 = _edge2(ei, g3.reshape(2 * N))
    out = _tc3(parts3, g3, p, Wc, bc)
    return out
